# baseline (speedup 1.0000x reference)
import jax
import jax.numpy as jnp
from jax.experimental import pallas as pl
from jax.experimental.pallas import tpu as pltpu

_K = 64
_ROWS_PER_BLOCK = 8
_N = 32768
_MIN_I32 = -2147483648
_BIG_I32 = 2147483647


def _topk_kernel(x_ref, res_ref, idx_ref):
    x = x_ref[...]
    b = pltpu.bitcast(x, jnp.int32)
    key = b ^ (jax.lax.shift_right_arithmetic(b, 31) & 0x7FFFFFFF)
    iota = jax.lax.broadcasted_iota(jnp.int32, (_ROWS_PER_BLOCK, _N), 1)
    lane64 = jax.lax.broadcasted_iota(jnp.int32, (_ROWS_PER_BLOCK, _K), 1)

    def body(r, carry):
        work, idxacc = carry
        m = jnp.max(work, axis=1, keepdims=True)
        cand = jnp.where(work == m, iota, _BIG_I32)
        gidx = jnp.min(cand, axis=1, keepdims=True)
        work = jnp.where(iota == gidx, _MIN_I32, work)
        idxacc = jnp.where(lane64 == r, gidx, idxacc)
        return work, idxacc

    work0 = key
    idx0 = jnp.zeros((_ROWS_PER_BLOCK, _K), jnp.int32)
    work, idxacc = jax.lax.fori_loop(0, _K, body, (work0, idx0))
    res_ref[...] = jnp.where(work == _MIN_I32, jnp.maximum(x, 0.0), 0.0)
    idx_ref[...] = idxacc


def kernel(x):
    rows, n = x.shape
    grid = (rows // _ROWS_PER_BLOCK,)
    result, idx = pl.pallas_call(
        _topk_kernel,
        grid=grid,
        in_specs=[pl.BlockSpec((_ROWS_PER_BLOCK, n), lambda i: (i, 0))],
        out_specs=[
            pl.BlockSpec((_ROWS_PER_BLOCK, n), lambda i: (i, 0)),
            pl.BlockSpec((_ROWS_PER_BLOCK, _K), lambda i: (i, 0)),
        ],
        out_shape=[
            jax.ShapeDtypeStruct((rows, n), x.dtype),
            jax.ShapeDtypeStruct((rows, _K), jnp.int32),
        ],
    )(x)
    return (result, idx)
